# baseline (speedup 1.0000x reference)
import numpy as np
import jax
import jax.numpy as jnp
from jax.experimental import pallas as pl
from jax.experimental.pallas import tpu as pltpu

_C4 = 16
_C = 64
_H = 16
_W = 16
_HW = _H * _W
_MIP = 8
_CR = 4
_BN = 32


def _cad_body(x1_ref, x2_ref, x3_ref, x4_ref, dz_ref, pool_ref, eh_ref,
              ew_ref, w1_ref, w2_ref, c1w_ref, c1b_ref, bns_ref, bnb_ref,
              chw_ref, chb_ref, cww_ref, cwb_ref, o_ref):
    f32 = jnp.float32
    HI = jax.lax.Precision.HIGHEST

    def dot(a, b):
        return jnp.dot(a, b, preferred_element_type=f32, precision=HI)

    def dot_t(a, b):
        return jax.lax.dot_general(a, b, (((1,), (1,)), ((), ())),
                                   preferred_element_type=f32, precision=HI)

    bn = x1_ref.shape[0]
    d = jax.nn.sigmoid(dz_ref[...])
    xs = (x1_ref[...], x2_ref[...], x3_ref[...], x4_ref[...])

    avgs, maxs = [], []
    for x in xs:
        xpd = x + d
        avgs.append(jnp.mean(xpd, axis=2))
        maxs.append(jnp.max(xpd, axis=2))
    ca_avg = jnp.concatenate(avgs, axis=1)
    ca_max = jnp.concatenate(maxs, axis=1)
    w1 = w1_ref[...]
    hmid = (jnp.maximum(dot_t(ca_avg, w1), 0.0)
            + jnp.maximum(dot_t(ca_max, w1), 0.0))
    ca = jax.nn.sigmoid(dot_t(hmid, w2_ref[...]))

    c1_wf = c1w_ref[...] * bns_ref[...]
    c1_bf = jnp.swapaxes(bns_ref[...] * c1b_ref[...] + bnb_ref[...], 0, 1)
    ip = xs[0] + xs[1] + xs[2] + xs[3] + d
    ypool = dot(ip.reshape(bn * _C4, _HW), pool_ref[...])
    ypt = jnp.swapaxes(ypool.reshape(bn, _C4, _H + _W), 1, 2)
    y = dot_t(ypt.reshape(bn * (_H + _W), _C4), c1_wf) + c1_bf
    y = y * (jnp.clip(y + 3.0, 0.0, 6.0) * (1.0 / 6.0))
    y3 = y.reshape(bn, _H + _W, _MIP)

    pre_h = dot_t(y3[:, 0:_H, :].reshape(bn * _H, _MIP), chw_ref[...]) \
        + jnp.swapaxes(chb_ref[...], 0, 1)
    pre_w = dot_t(y3[:, _H:, :].reshape(bn * _W, _MIP), cww_ref[...]) \
        + jnp.swapaxes(cwb_ref[...], 0, 1)
    at = jax.nn.sigmoid(jnp.concatenate(
        [pre_h.reshape(bn, _H, _C4), pre_w.reshape(bn, _W, _C4)], axis=1))
    a = jnp.swapaxes(at, 1, 2)

    ah = dot(a[:, :, 0:_H].reshape(bn * _C4, _H), eh_ref[...])
    aw = dot(a[:, :, _H:].reshape(bn * _C4, _W), ew_ref[...])
    hw_a = ip * (ah * aw).reshape(bn, _C4, _HW)

    for i in range(4):
        o_ref[:, i * _C4:(i + 1) * _C4, :] = (
            ca[:, i * _C4:(i + 1) * _C4, None] * (xs[i] + hw_a))


def kernel(x1, x2, x3, x4, distance, ca_w1, ca_w2, c1_w, c1_b, bn_scale,
           bn_shift, ch_w, ch_b, cw_w, cw_b):
    f32 = jnp.float32
    n = x1.shape[0]

    x1f = x1.reshape(n, _C4, _HW)
    x2f = x2.reshape(n, _C4, _HW)
    x3f = x3.reshape(n, _C4, _HW)
    x4f = x4.reshape(n, _C4, _HW)
    dzf = distance.reshape(n, 1, _HW)

    l = np.arange(_HW)
    eh = (l[None, :] // _W == np.arange(_H)[:, None]).astype(np.float32)
    ew = (l[None, :] % _W == np.arange(_W)[:, None]).astype(np.float32)
    pool = np.concatenate([eh.T / _W, ew.T / _H], axis=1)

    const = lambda shape: pl.BlockSpec(shape, lambda i: (0,) * len(shape))
    out_flat = pl.pallas_call(
        _cad_body,
        grid=(n // _BN,),
        in_specs=[
            pl.BlockSpec((_BN, _C4, _HW), lambda i: (i, 0, 0)),
            pl.BlockSpec((_BN, _C4, _HW), lambda i: (i, 0, 0)),
            pl.BlockSpec((_BN, _C4, _HW), lambda i: (i, 0, 0)),
            pl.BlockSpec((_BN, _C4, _HW), lambda i: (i, 0, 0)),
            pl.BlockSpec((_BN, 1, _HW), lambda i: (i, 0, 0)),
            const((_HW, _H + _W)),
            const((_H, _HW)),
            const((_W, _HW)),
            const((_CR, _C)),
            const((_C, _CR)),
            const((_MIP, _C4)),
            const((_MIP, 1)),
            const((_MIP, 1)),
            const((_MIP, 1)),
            const((_C4, _MIP)),
            const((_C4, 1)),
            const((_C4, _MIP)),
            const((_C4, 1)),
        ],
        out_specs=pl.BlockSpec((_BN, _C, _HW), lambda i: (i, 0, 0)),
        out_shape=jax.ShapeDtypeStruct((n, _C, _HW), f32),
        compiler_params=pltpu.CompilerParams(
            dimension_semantics=("parallel",)),
    )(x1f, x2f, x3f, x4f, dzf, jnp.asarray(pool), jnp.asarray(eh),
      jnp.asarray(ew), ca_w1, ca_w2, c1_w, c1_b, bn_scale, bn_shift,
      ch_w, ch_b, cw_w, cw_b)

    return out_flat.reshape(n, _C, _H, _W)


# speedup vs baseline: 1.2555x; 1.2555x over previous
import numpy as np
import jax
import jax.numpy as jnp
from jax.experimental import pallas as pl
from jax.experimental.pallas import tpu as pltpu

_C4 = 16
_C = 64
_H = 16
_W = 16
_HW = _H * _W
_MIP = 8
_CR = 4
_BN = 32


def _cad_body(x1_ref, x2_ref, x3_ref, x4_ref, dz_ref, pool_ref, eh_ref,
              ew_ref, w1_ref, w2_ref, c1w_ref, c1b_ref, bns_ref, bnb_ref,
              chw_ref, chb_ref, cww_ref, cwb_ref, o_ref):
    f32 = jnp.float32

    def dot(a, b):
        return jnp.dot(a, b, preferred_element_type=f32)

    def dot_t(a, b):
        return jax.lax.dot_general(a, b, (((1,), (1,)), ((), ())),
                                   preferred_element_type=f32)

    bn = x1_ref.shape[0]
    d = jax.nn.sigmoid(dz_ref[...])
    xs = (x1_ref[...], x2_ref[...], x3_ref[...], x4_ref[...])

    avgs, maxs = [], []
    for x in xs:
        xpd = x + d
        avgs.append(jnp.mean(xpd, axis=2))
        maxs.append(jnp.max(xpd, axis=2))
    ca_avg = jnp.concatenate(avgs, axis=1)
    ca_max = jnp.concatenate(maxs, axis=1)
    w1 = w1_ref[...]
    hmid = (jnp.maximum(dot_t(ca_avg, w1), 0.0)
            + jnp.maximum(dot_t(ca_max, w1), 0.0))
    ca = jax.nn.sigmoid(dot_t(hmid, w2_ref[...]))

    c1_wf = c1w_ref[...] * bns_ref[...]
    c1_bf = jnp.swapaxes(bns_ref[...] * c1b_ref[...] + bnb_ref[...], 0, 1)
    ip = xs[0] + xs[1] + xs[2] + xs[3] + d
    ypool = dot(ip.reshape(bn * _C4, _HW), pool_ref[...])
    ypt = jnp.swapaxes(ypool.reshape(bn, _C4, _H + _W), 1, 2)
    y = dot_t(ypt.reshape(bn * (_H + _W), _C4), c1_wf) + c1_bf
    y = y * (jnp.clip(y + 3.0, 0.0, 6.0) * (1.0 / 6.0))
    y3 = y.reshape(bn, _H + _W, _MIP)

    pre_h = dot_t(y3[:, 0:_H, :].reshape(bn * _H, _MIP), chw_ref[...]) \
        + jnp.swapaxes(chb_ref[...], 0, 1)
    pre_w = dot_t(y3[:, _H:, :].reshape(bn * _W, _MIP), cww_ref[...]) \
        + jnp.swapaxes(cwb_ref[...], 0, 1)
    at = jax.nn.sigmoid(jnp.concatenate(
        [pre_h.reshape(bn, _H, _C4), pre_w.reshape(bn, _W, _C4)], axis=1))
    a = jnp.swapaxes(at, 1, 2)

    ah = dot(a[:, :, 0:_H].reshape(bn * _C4, _H), eh_ref[...])
    aw = dot(a[:, :, _H:].reshape(bn * _C4, _W), ew_ref[...])
    hw_a = ip * (ah * aw).reshape(bn, _C4, _HW)

    for i in range(4):
        o_ref[:, i * _C4:(i + 1) * _C4, :] = (
            ca[:, i * _C4:(i + 1) * _C4, None] * (xs[i] + hw_a))


def kernel(x1, x2, x3, x4, distance, ca_w1, ca_w2, c1_w, c1_b, bn_scale,
           bn_shift, ch_w, ch_b, cw_w, cw_b):
    f32 = jnp.float32
    n = x1.shape[0]

    x1f = x1.reshape(n, _C4, _HW)
    x2f = x2.reshape(n, _C4, _HW)
    x3f = x3.reshape(n, _C4, _HW)
    x4f = x4.reshape(n, _C4, _HW)
    dzf = distance.reshape(n, 1, _HW)

    l = np.arange(_HW)
    eh = (l[None, :] // _W == np.arange(_H)[:, None]).astype(np.float32)
    ew = (l[None, :] % _W == np.arange(_W)[:, None]).astype(np.float32)
    pool = np.concatenate([eh.T / _W, ew.T / _H], axis=1)

    bidx = lambda i: (i, 0, 0)
    const = lambda shape: pl.BlockSpec(shape, lambda i: (0,) * len(shape))
    out_flat = pl.pallas_call(
        _cad_body,
        grid=(n // _BN,),
        in_specs=[
            pl.BlockSpec((_BN, _C4, _HW), bidx),
            pl.BlockSpec((_BN, _C4, _HW), bidx),
            pl.BlockSpec((_BN, _C4, _HW), bidx),
            pl.BlockSpec((_BN, _C4, _HW), bidx),
            pl.BlockSpec((_BN, 1, _HW), bidx),
            const((_HW, _H + _W)),
            const((_H, _HW)),
            const((_W, _HW)),
            const((_CR, _C)),
            const((_C, _CR)),
            const((_MIP, _C4)),
            const((_MIP, 1)),
            const((_MIP, 1)),
            const((_MIP, 1)),
            const((_C4, _MIP)),
            const((_C4, 1)),
            const((_C4, _MIP)),
            const((_C4, 1)),
        ],
        out_specs=pl.BlockSpec((_BN, _C, _HW), bidx),
        out_shape=jax.ShapeDtypeStruct((n, _C, _HW), f32),
        compiler_params=pltpu.CompilerParams(
            dimension_semantics=("arbitrary",)),
    )(x1f, x2f, x3f, x4f, dzf, jnp.asarray(pool), jnp.asarray(eh),
      jnp.asarray(ew), ca_w1, ca_w2, c1_w, c1_b, bn_scale, bn_shift,
      ch_w, ch_b, cw_w, cw_b)

    return out_flat.reshape(n, _C, _H, _W)


# speedup vs baseline: 1.3735x; 1.0939x over previous
import numpy as np
import jax
import jax.numpy as jnp
from jax.experimental import pallas as pl
from jax.experimental.pallas import tpu as pltpu

_C4 = 16
_C = 64
_H = 16
_W = 16
_HW = _H * _W
_MIP = 8
_CR = 4
_BN = 32


def _cad_body(x1_ref, x2_ref, x3_ref, x4_ref, dz_ref, pool_ref, eh_ref,
              ew_ref, w1t_ref, w2t_ref, c1t_ref, c1b_ref, cht_ref, chb_ref,
              cwt_ref, cwb_ref, o_ref):
    f32 = jnp.float32

    def dot(a, b):
        return jnp.dot(a, b, preferred_element_type=f32)

    bn = x1_ref.shape[0]
    d = jax.nn.sigmoid(dz_ref[...])
    xs = (x1_ref[...], x2_ref[...], x3_ref[...], x4_ref[...])

    avgs, maxs = [], []
    for x in xs:
        xpd = x + d
        avgs.append(jnp.mean(xpd, axis=2))
        maxs.append(jnp.max(xpd, axis=2))
    ca_avg = jnp.concatenate(avgs, axis=1)
    ca_max = jnp.concatenate(maxs, axis=1)
    w1t = w1t_ref[...]
    hmid = (jnp.maximum(dot(ca_avg, w1t), 0.0)
            + jnp.maximum(dot(ca_max, w1t), 0.0))
    ca = jax.nn.sigmoid(dot(hmid, w2t_ref[...]))

    ip = xs[0] + xs[1] + xs[2] + xs[3] + d
    ypool = dot(ip.reshape(bn * _C4, _HW), pool_ref[...])
    ypt = jnp.swapaxes(ypool.reshape(bn, _C4, _H + _W), 1, 2)
    y = dot(ypt.reshape(bn * (_H + _W), _C4), c1t_ref[...]) + c1b_ref[...]
    y = y * (jnp.clip(y + 3.0, 0.0, 6.0) * (1.0 / 6.0))
    y3 = y.reshape(bn, _H + _W, _MIP)

    pre_h = dot(y3[:, 0:_H, :].reshape(bn * _H, _MIP), cht_ref[...]) \
        + chb_ref[...]
    pre_w = dot(y3[:, _H:, :].reshape(bn * _W, _MIP), cwt_ref[...]) \
        + cwb_ref[...]
    at = jax.nn.sigmoid(jnp.concatenate(
        [pre_h.reshape(bn, _H, _C4), pre_w.reshape(bn, _W, _C4)], axis=1))
    a = jnp.swapaxes(at, 1, 2)

    ah = dot(a[:, :, 0:_H].reshape(bn * _C4, _H), eh_ref[...])
    aw = dot(a[:, :, _H:].reshape(bn * _C4, _W), ew_ref[...])
    hw_a = ip * (ah * aw).reshape(bn, _C4, _HW)

    for i in range(4):
        o_ref[:, i * _C4:(i + 1) * _C4, :] = (
            ca[:, i * _C4:(i + 1) * _C4, None] * (xs[i] + hw_a))


def kernel(x1, x2, x3, x4, distance, ca_w1, ca_w2, c1_w, c1_b, bn_scale,
           bn_shift, ch_w, ch_b, cw_w, cw_b):
    f32 = jnp.float32
    n = x1.shape[0]

    x1f = x1.reshape(n, _C4, _HW)
    x2f = x2.reshape(n, _C4, _HW)
    x3f = x3.reshape(n, _C4, _HW)
    x4f = x4.reshape(n, _C4, _HW)
    dzf = distance.reshape(n, 1, _HW)

    l = np.arange(_HW)
    eh = (l[None, :] // _W == np.arange(_H)[:, None]).astype(np.float32)
    ew = (l[None, :] % _W == np.arange(_W)[:, None]).astype(np.float32)
    pool = np.concatenate([eh.T / _W, ew.T / _H], axis=1)

    c1_wf = c1_w * bn_scale
    c1_bf = bn_scale * c1_b + bn_shift
    w1t = ca_w1.T
    w2t = ca_w2.T
    c1t = c1_wf.T
    c1b = c1_bf.T
    cht = ch_w.T
    chb = ch_b.T
    cwt = cw_w.T
    cwb = cw_b.T

    bidx = lambda i: (i, 0, 0)
    const = lambda shape: pl.BlockSpec(shape, lambda i: (0,) * len(shape))
    out_flat = pl.pallas_call(
        _cad_body,
        grid=(n // _BN,),
        in_specs=[
            pl.BlockSpec((_BN, _C4, _HW), bidx),
            pl.BlockSpec((_BN, _C4, _HW), bidx),
            pl.BlockSpec((_BN, _C4, _HW), bidx),
            pl.BlockSpec((_BN, _C4, _HW), bidx),
            pl.BlockSpec((_BN, 1, _HW), bidx),
            const((_HW, _H + _W)),
            const((_H, _HW)),
            const((_W, _HW)),
            const((_C, _CR)),
            const((_CR, _C)),
            const((_C4, _MIP)),
            const((1, _MIP)),
            const((_MIP, _C4)),
            const((1, _C4)),
            const((_MIP, _C4)),
            const((1, _C4)),
        ],
        out_specs=pl.BlockSpec((_BN, _C, _HW), bidx),
        out_shape=jax.ShapeDtypeStruct((n, _C, _HW), f32),
        compiler_params=pltpu.CompilerParams(
            dimension_semantics=("arbitrary",)),
    )(x1f, x2f, x3f, x4f, dzf, jnp.asarray(pool), jnp.asarray(eh),
      jnp.asarray(ew), w1t, w2t, c1t, c1b, cht, chb, cwt, cwb)

    return out_flat.reshape(n, _C, _H, _W)


# speedup vs baseline: 1.4324x; 1.0429x over previous
import numpy as np
import jax
import jax.numpy as jnp
from jax.experimental import pallas as pl
from jax.experimental.pallas import tpu as pltpu

_C4 = 16
_C = 64
_H = 16
_W = 16
_HW = _H * _W
_MIP = 8
_CR = 4
_BN = 64


def _cad_body(x1_ref, x2_ref, x3_ref, x4_ref, dz_ref, pool_ref, eh_ref,
              ew_ref, w1t_ref, w2t_ref, c1t_ref, c1b_ref, cht_ref, chb_ref,
              cwt_ref, cwb_ref, o_ref):
    f32 = jnp.float32

    def dot(a, b):
        return jnp.dot(a, b, preferred_element_type=f32)

    bn = x1_ref.shape[0]
    d = jax.nn.sigmoid(dz_ref[...])
    xs = (x1_ref[...], x2_ref[...], x3_ref[...], x4_ref[...])

    avgs, maxs = [], []
    for x in xs:
        xpd = x + d
        avgs.append(jnp.mean(xpd, axis=2))
        maxs.append(jnp.max(xpd, axis=2))
    ca_avg = jnp.concatenate(avgs, axis=1)
    ca_max = jnp.concatenate(maxs, axis=1)
    w1t = w1t_ref[...]
    hmid = (jnp.maximum(dot(ca_avg, w1t), 0.0)
            + jnp.maximum(dot(ca_max, w1t), 0.0))
    ca = jax.nn.sigmoid(dot(hmid, w2t_ref[...]))

    ip = xs[0] + xs[1] + xs[2] + xs[3] + d
    ypool = dot(ip.reshape(bn * _C4, _HW), pool_ref[...])
    ypt = jnp.swapaxes(ypool.reshape(bn, _C4, _H + _W), 1, 2)
    y = dot(ypt.reshape(bn * (_H + _W), _C4), c1t_ref[...]) + c1b_ref[...]
    y = y * (jnp.clip(y + 3.0, 0.0, 6.0) * (1.0 / 6.0))
    y3 = y.reshape(bn, _H + _W, _MIP)

    pre_h = dot(y3[:, 0:_H, :].reshape(bn * _H, _MIP), cht_ref[...]) \
        + chb_ref[...]
    pre_w = dot(y3[:, _H:, :].reshape(bn * _W, _MIP), cwt_ref[...]) \
        + cwb_ref[...]
    at = jax.nn.sigmoid(jnp.concatenate(
        [pre_h.reshape(bn, _H, _C4), pre_w.reshape(bn, _W, _C4)], axis=1))
    a = jnp.swapaxes(at, 1, 2)

    ah = dot(a[:, :, 0:_H].reshape(bn * _C4, _H), eh_ref[...])
    aw = dot(a[:, :, _H:].reshape(bn * _C4, _W), ew_ref[...])
    hw_a = ip * (ah * aw).reshape(bn, _C4, _HW)

    for i in range(4):
        o_ref[:, i * _C4:(i + 1) * _C4, :] = (
            ca[:, i * _C4:(i + 1) * _C4, None] * (xs[i] + hw_a))


def kernel(x1, x2, x3, x4, distance, ca_w1, ca_w2, c1_w, c1_b, bn_scale,
           bn_shift, ch_w, ch_b, cw_w, cw_b):
    f32 = jnp.float32
    n = x1.shape[0]

    x1f = x1.reshape(n, _C4, _HW)
    x2f = x2.reshape(n, _C4, _HW)
    x3f = x3.reshape(n, _C4, _HW)
    x4f = x4.reshape(n, _C4, _HW)
    dzf = distance.reshape(n, 1, _HW)

    l = np.arange(_HW)
    eh = (l[None, :] // _W == np.arange(_H)[:, None]).astype(np.float32)
    ew = (l[None, :] % _W == np.arange(_W)[:, None]).astype(np.float32)
    pool = np.concatenate([eh.T / _W, ew.T / _H], axis=1)

    c1_wf = c1_w * bn_scale
    c1_bf = bn_scale * c1_b + bn_shift
    w1t = ca_w1.T
    w2t = ca_w2.T
    c1t = c1_wf.T
    c1b = c1_bf.T
    cht = ch_w.T
    chb = ch_b.T
    cwt = cw_w.T
    cwb = cw_b.T

    bidx = lambda i: (i, 0, 0)
    const = lambda shape: pl.BlockSpec(shape, lambda i: (0,) * len(shape))
    out_flat = pl.pallas_call(
        _cad_body,
        grid=(n // _BN,),
        in_specs=[
            pl.BlockSpec((_BN, _C4, _HW), bidx),
            pl.BlockSpec((_BN, _C4, _HW), bidx),
            pl.BlockSpec((_BN, _C4, _HW), bidx),
            pl.BlockSpec((_BN, _C4, _HW), bidx),
            pl.BlockSpec((_BN, 1, _HW), bidx),
            const((_HW, _H + _W)),
            const((_H, _HW)),
            const((_W, _HW)),
            const((_C, _CR)),
            const((_CR, _C)),
            const((_C4, _MIP)),
            const((1, _MIP)),
            const((_MIP, _C4)),
            const((1, _C4)),
            const((_MIP, _C4)),
            const((1, _C4)),
        ],
        out_specs=pl.BlockSpec((_BN, _C, _HW), bidx),
        out_shape=jax.ShapeDtypeStruct((n, _C, _HW), f32),
        compiler_params=pltpu.CompilerParams(
            dimension_semantics=("arbitrary",)),
    )(x1f, x2f, x3f, x4f, dzf, jnp.asarray(pool), jnp.asarray(eh),
      jnp.asarray(ew), w1t, w2t, c1t, c1b, cht, chb, cwt, cwb)

    return out_flat.reshape(n, _C, _H, _W)
